# baseline (speedup 1.0000x reference)
import functools

import jax
import jax.numpy as jnp
from jax import lax
from jax.experimental import pallas as pl
from jax.experimental.pallas import tpu as pltpu
from jax.experimental.pallas import tpu_sc as plsc

NC = 2
NS = 16
NW = NC * NS
LANES = 16

VP = 1024
SPAD = 64
TPAD = 32

_HIGH = jax.lax.Precision.HIGHEST


def _mesh():
    return plsc.VectorSubcoreMesh(
        core_axis_name="c", subcore_axis_name="s", num_cores=NC, num_subcores=NS)


def _wid():
    return lax.axis_index("s") * NC + lax.axis_index("c")


def _make_hop(B, story, T):
    rpw = B // NW
    nvec = SPAD // LANES
    n_real_last = story - (nvec - 1) * LANES

    @functools.partial(
        pl.kernel,
        mesh=_mesh(),
        out_type=jax.ShapeDtypeStruct((B, VP), jnp.float32),
        scratch_types=[
            pltpu.VMEM((rpw, VP), jnp.float32),
            pltpu.VMEM((rpw, T, SPAD), jnp.int32),
            pltpu.VMEM((rpw, VP), jnp.float32),
            pltpu.SemaphoreType.DMA,
            pltpu.SemaphoreType.DMA,
            pltpu.SemaphoreType.DMA,
        ],
        compiler_params=pltpu.CompilerParams(use_tc_tiling_on_sc=False, needs_layout_passes=False),
    )
    def hop(s_hbm, x_hbm, z_hbm, w_hbm, s_v, x_v, w_v, sem_s, sem_x, sem_z):
        base = _wid() * rpw
        cp_s = pltpu.async_copy(s_hbm.at[pl.ds(base, rpw)], s_v, sem_s)
        cp_x = pltpu.async_copy(x_hbm.at[pl.ds(base, rpw)], x_v, sem_x)
        cp_z = pltpu.async_copy(z_hbm, w_v, sem_z)
        cp_s.wait()
        cp_x.wait()
        cp_z.wait()

        lane = lax.iota(jnp.int32, LANES)
        neg_inf = jnp.full((LANES,), -jnp.inf, jnp.float32)

        def row(r, carry):
            rvec = jnp.full((LANES,), r, jnp.int32)
            accs = [jnp.zeros((LANES,), jnp.float32) for _ in range(nvec)]
            for t in range(T):
                for v in range(nvec):
                    idx = x_v[r, t, pl.ds(v * LANES, LANES)]
                    accs[v] = accs[v] + plsc.load_gather(s_v, [rvec, idx])
            accs[-1] = jnp.where(lane < n_real_last, accs[-1], neg_inf)
            m = accs[0]
            for v in range(1, nvec):
                m = jnp.maximum(m, accs[v])
            mx = jnp.max(m)
            es = [jnp.exp(a - jnp.full((LANES,), mx, jnp.float32)) for a in accs]
            tot = es[0]
            for v in range(1, nvec):
                tot = tot + es[v]
            totv = jnp.full((LANES,), jnp.sum(tot), jnp.float32)
            ps = [e / totv for e in es]
            for t in range(T):
                for v in range(nvec):
                    idx = x_v[r, t, pl.ds(v * LANES, LANES)]
                    plsc.addupdate_scatter(w_v, [rvec, idx], ps[v])
            return carry

        lax.fori_loop(0, rpw, row, 0)
        pltpu.sync_copy(w_v, w_hbm.at[pl.ds(base, rpw)])

    return hop


def _make_count(B):
    rpw = B // NW
    nvec = TPAD // LANES

    @functools.partial(
        pl.kernel,
        mesh=_mesh(),
        out_type=jax.ShapeDtypeStruct((B, VP), jnp.float32),
        scratch_types=[
            pltpu.VMEM((rpw, TPAD), jnp.int32),
            pltpu.VMEM((rpw, VP), jnp.float32),
            pltpu.SemaphoreType.DMA,
            pltpu.SemaphoreType.DMA,
        ],
        compiler_params=pltpu.CompilerParams(use_tc_tiling_on_sc=False, needs_layout_passes=False),
    )
    def count(q_hbm, z_hbm, c_hbm, q_v, c_v, sem_q, sem_z):
        base = _wid() * rpw
        cp_q = pltpu.async_copy(q_hbm.at[pl.ds(base, rpw)], q_v, sem_q)
        cp_z = pltpu.async_copy(z_hbm, c_v, sem_z)
        cp_q.wait()
        cp_z.wait()
        ones = jnp.full((LANES,), 1.0, jnp.float32)

        def row(r, carry):
            rvec = jnp.full((LANES,), r, jnp.int32)
            for v in range(nvec):
                idx = q_v[r, pl.ds(v * LANES, LANES)]
                plsc.addupdate_scatter(c_v, [rvec, idx], ones)
            return carry

        lax.fori_loop(0, rpw, row, 0)
        pltpu.sync_copy(c_v, c_hbm.at[pl.ds(base, rpw)])

    return count


def _stage_body(w_ref, u_ref, a_ref, s_ref, unew_ref):
    a = a_ref[...]
    u = u_ref[...] + jax.lax.dot(
        w_ref[...], a, precision=_HIGH, preferred_element_type=jnp.float32)
    unew_ref[...] = u
    s_ref[...] = jax.lax.dot_general(
        u, a, (((1,), (1,)), ((), ())), precision=_HIGH,
        preferred_element_type=jnp.float32)


def _final_body(v_real, w_ref, u_ref, a_ref, out_ref):
    a = a_ref[...]
    u = u_ref[...] + jax.lax.dot(
        w_ref[...], a, precision=_HIGH, preferred_element_type=jnp.float32)
    logits = jax.lax.dot_general(
        u, a, (((1,), (1,)), ((), ())), precision=_HIGH,
        preferred_element_type=jnp.float32)
    col = lax.broadcasted_iota(jnp.int32, logits.shape, 1)
    logits = jnp.where(col < v_real, logits, -jnp.inf)
    mx = jnp.max(logits, axis=1, keepdims=True)
    e = jnp.exp(logits - mx)
    p = e / jnp.sum(e, axis=1, keepdims=True)
    out_ref[...] = lax.slice(p, (0, 0), (p.shape[0], v_real))


def kernel(x, query, A0, A1, A2, A3):
    B, story, sent = x.shape
    _, qlen = query.shape
    V, E = A0.shape

    xT = jnp.transpose(x, (0, 2, 1))
    xT = jnp.pad(xT, ((0, 0), (0, 0), (0, SPAD - story)))
    qp = jnp.pad(query, ((0, 0), (0, TPAD - qlen)))
    Ap = [jnp.pad(A, ((0, VP - V), (0, 0))) for A in (A0, A1, A2, A3)]

    hop = _make_hop(B, story, sent)
    count = _make_count(B)
    stage = pl.pallas_call(
        _stage_body,
        out_shape=[
            jax.ShapeDtypeStruct((B, VP), jnp.float32),
            jax.ShapeDtypeStruct((B, E), jnp.float32),
        ],
    )
    final = pl.pallas_call(
        functools.partial(_final_body, V),
        out_shape=jax.ShapeDtypeStruct((B, V), jnp.float32),
    )

    rpw = B // NW
    zeros = jnp.zeros((rpw, VP), jnp.float32)
    Cq = count(qp, zeros)
    u = jnp.zeros((B, E), jnp.float32)
    W = Cq
    for k in range(3):
        S, u = stage(W, u, Ap[k])
        W = hop(S, xT, zeros)
    return final(W, u, Ap[3])


# speedup vs baseline: 1.0569x; 1.0569x over previous
import functools

import jax
import jax.numpy as jnp
from jax import lax
from jax.experimental import pallas as pl
from jax.experimental.pallas import tpu as pltpu
from jax.experimental.pallas import tpu_sc as plsc

NC = 2
NS = 16
NW = NC * NS
LANES = 16

VP = 1024
SPAD = 64
TPAD = 32

_HIGH = jax.lax.Precision.HIGHEST


def _mesh():
    return plsc.VectorSubcoreMesh(
        core_axis_name="c", subcore_axis_name="s", num_cores=NC, num_subcores=NS)


def _wid():
    return lax.axis_index("s") * NC + lax.axis_index("c")


def _make_hop(B, story, T):
    rpw = B // NW
    nvec = SPAD // LANES
    n_real_last = story - (nvec - 1) * LANES

    @functools.partial(
        pl.kernel,
        mesh=_mesh(),
        out_type=jax.ShapeDtypeStruct((B, VP), jnp.float32),
        scratch_types=[
            pltpu.VMEM((rpw, VP), jnp.float32),
            pltpu.VMEM((rpw, T, SPAD), jnp.int32),
            pltpu.VMEM((rpw, VP), jnp.float32),
            pltpu.SemaphoreType.DMA,
            pltpu.SemaphoreType.DMA,
        ],
        compiler_params=pltpu.CompilerParams(use_tc_tiling_on_sc=False, needs_layout_passes=False),
    )
    def hop(s_hbm, x_hbm, w_hbm, s_v, x_v, w_v, sem_s, sem_x):
        base = _wid() * rpw
        cp_s = pltpu.async_copy(s_hbm.at[pl.ds(base, rpw)], s_v, sem_s)
        cp_x = pltpu.async_copy(x_hbm.at[pl.ds(base, rpw)], x_v, sem_x)
        cp_s.wait()
        cp_x.wait()

        lane = lax.iota(jnp.int32, LANES)
        neg_inf = jnp.full((LANES,), -jnp.inf, jnp.float32)

        def row(r, carry):
            for c in range(VP // LANES):
                w_v[r, pl.ds(c * LANES, LANES)] = jnp.zeros((LANES,), jnp.float32)
            rvec = jnp.full((LANES,), r, jnp.int32)
            accs = [jnp.zeros((LANES,), jnp.float32) for _ in range(nvec)]
            for t in range(T):
                for v in range(nvec):
                    idx = x_v[r, t, pl.ds(v * LANES, LANES)]
                    accs[v] = accs[v] + plsc.load_gather(s_v, [rvec, idx])
            accs[-1] = jnp.where(lane < n_real_last, accs[-1], neg_inf)
            m = accs[0]
            for v in range(1, nvec):
                m = jnp.maximum(m, accs[v])
            mx = jnp.max(m)
            es = [jnp.exp(a - jnp.full((LANES,), mx, jnp.float32)) for a in accs]
            tot = es[0]
            for v in range(1, nvec):
                tot = tot + es[v]
            totv = jnp.full((LANES,), jnp.sum(tot), jnp.float32)
            ps = [e / totv for e in es]
            for t in range(T):
                for v in range(nvec):
                    idx = x_v[r, t, pl.ds(v * LANES, LANES)]
                    plsc.addupdate_scatter(w_v, [rvec, idx], ps[v])
            return carry

        lax.fori_loop(0, rpw, row, 0)
        pltpu.sync_copy(w_v, w_hbm.at[pl.ds(base, rpw)])

    return hop


def _make_count(B):
    rpw = B // NW
    nvec = TPAD // LANES

    @functools.partial(
        pl.kernel,
        mesh=_mesh(),
        out_type=jax.ShapeDtypeStruct((B, VP), jnp.float32),
        scratch_types=[
            pltpu.VMEM((rpw, TPAD), jnp.int32),
            pltpu.VMEM((rpw, VP), jnp.float32),
        ],
        compiler_params=pltpu.CompilerParams(use_tc_tiling_on_sc=False, needs_layout_passes=False),
    )
    def count(q_hbm, c_hbm, q_v, c_v):
        base = _wid() * rpw
        pltpu.sync_copy(q_hbm.at[pl.ds(base, rpw)], q_v)
        ones = jnp.full((LANES,), 1.0, jnp.float32)

        def row(r, carry):
            for c in range(VP // LANES):
                c_v[r, pl.ds(c * LANES, LANES)] = jnp.zeros((LANES,), jnp.float32)
            rvec = jnp.full((LANES,), r, jnp.int32)
            for v in range(nvec):
                idx = q_v[r, pl.ds(v * LANES, LANES)]
                plsc.addupdate_scatter(c_v, [rvec, idx], ones)
            return carry

        lax.fori_loop(0, rpw, row, 0)
        pltpu.sync_copy(c_v, c_hbm.at[pl.ds(base, rpw)])

    return count


def _stage_body(w_ref, u_ref, a_ref, s_ref, unew_ref):
    a = a_ref[...]
    u = u_ref[...] + jax.lax.dot(
        w_ref[...], a, precision=_HIGH, preferred_element_type=jnp.float32)
    unew_ref[...] = u
    s_ref[...] = jax.lax.dot_general(
        u, a, (((1,), (1,)), ((), ())), precision=_HIGH,
        preferred_element_type=jnp.float32)


def _final_body(v_real, w_ref, u_ref, a_ref, out_ref):
    a = a_ref[...]
    u = u_ref[...] + jax.lax.dot(
        w_ref[...], a, precision=_HIGH, preferred_element_type=jnp.float32)
    logits = jax.lax.dot_general(
        u, a, (((1,), (1,)), ((), ())), precision=_HIGH,
        preferred_element_type=jnp.float32)
    col = lax.broadcasted_iota(jnp.int32, logits.shape, 1)
    logits = jnp.where(col < v_real, logits, -jnp.inf)
    mx = jnp.max(logits, axis=1, keepdims=True)
    e = jnp.exp(logits - mx)
    p = e / jnp.sum(e, axis=1, keepdims=True)
    out_ref[...] = lax.slice(p, (0, 0), (p.shape[0], v_real))


def kernel(x, query, A0, A1, A2, A3):
    B, story, sent = x.shape
    _, qlen = query.shape
    V, E = A0.shape

    xT = jnp.transpose(x, (0, 2, 1))
    xT = jnp.pad(xT, ((0, 0), (0, 0), (0, SPAD - story)))
    qp = jnp.pad(query, ((0, 0), (0, TPAD - qlen)))
    Ap = [jnp.pad(A, ((0, VP - V), (0, 0))) for A in (A0, A1, A2, A3)]

    hop = _make_hop(B, story, sent)
    count = _make_count(B)
    stage = pl.pallas_call(
        _stage_body,
        out_shape=[
            jax.ShapeDtypeStruct((B, VP), jnp.float32),
            jax.ShapeDtypeStruct((B, E), jnp.float32),
        ],
    )
    final = pl.pallas_call(
        functools.partial(_final_body, V),
        out_shape=jax.ShapeDtypeStruct((B, V), jnp.float32),
    )

    Cq = count(qp)
    u = jnp.zeros((B, E), jnp.float32)
    W = Cq
    for k in range(3):
        S, u = stage(W, u, Ap[k])
        W = hop(S, xT)
    return final(W, u, Ap[3])


# speedup vs baseline: 1.1575x; 1.0952x over previous
import functools

import jax
import jax.numpy as jnp
from jax import lax
from jax.experimental import pallas as pl
from jax.experimental.pallas import tpu as pltpu
from jax.experimental.pallas import tpu_sc as plsc

NC = 2
NS = 16
NW = NC * NS
LANES = 16

VP = 1024
SPAD = 64
TPAD = 32

_HIGH = jax.lax.Precision.HIGHEST


def _mesh():
    return plsc.VectorSubcoreMesh(
        core_axis_name="c", subcore_axis_name="s", num_cores=NC, num_subcores=NS)


def _wid():
    return lax.axis_index("s") * NC + lax.axis_index("c")


def _make_hop(B, story, T):
    rpw = B // NW
    nvec = SPAD // LANES
    n_real_last = story - (nvec - 1) * LANES
    xw = T * SPAD

    @functools.partial(
        pl.kernel,
        mesh=_mesh(),
        out_type=jax.ShapeDtypeStruct((B * VP,), jnp.float32),
        scratch_types=[
            pltpu.VMEM((rpw * VP,), jnp.float32),
            pltpu.VMEM((rpw * xw,), jnp.int32),
            pltpu.VMEM((rpw * VP,), jnp.float32),
            pltpu.SemaphoreType.DMA,
            pltpu.SemaphoreType.DMA,
        ],
        compiler_params=pltpu.CompilerParams(use_tc_tiling_on_sc=False, needs_layout_passes=False),
    )
    def hop(s_hbm, x_hbm, w_hbm, s_v, x_v, w_v, sem_s, sem_x):
        w = _wid()
        cp_s = pltpu.async_copy(s_hbm.at[pl.ds(w * rpw * VP, rpw * VP)], s_v, sem_s)
        cp_x = pltpu.async_copy(x_hbm.at[pl.ds(w * rpw * xw, rpw * xw)], x_v, sem_x)
        cp_s.wait()
        cp_x.wait()

        lane = lax.iota(jnp.int32, LANES)
        neg_inf = jnp.full((LANES,), -jnp.inf, jnp.float32)

        def row(r, carry):
            wbase = r * VP
            xbase = r * xw
            for c in range(VP // LANES):
                w_v[pl.ds(wbase + c * LANES, LANES)] = jnp.zeros((LANES,), jnp.float32)
            accs = [jnp.zeros((LANES,), jnp.float32) for _ in range(nvec)]
            for t in range(T):
                for v in range(nvec):
                    idx = x_v[pl.ds(xbase + t * SPAD + v * LANES, LANES)]
                    accs[v] = accs[v] + plsc.load_gather(s_v, [idx])
            accs[-1] = jnp.where(lane < n_real_last, accs[-1], neg_inf)
            m = accs[0]
            for v in range(1, nvec):
                m = jnp.maximum(m, accs[v])
            mx = jnp.max(m)
            es = [jnp.exp(a - jnp.full((LANES,), mx, jnp.float32)) for a in accs]
            tot = es[0]
            for v in range(1, nvec):
                tot = tot + es[v]
            totv = jnp.full((LANES,), jnp.sum(tot), jnp.float32)
            ps = [e / totv for e in es]
            for t in range(T):
                for v in range(nvec):
                    idx = x_v[pl.ds(xbase + t * SPAD + v * LANES, LANES)]
                    plsc.addupdate_scatter(w_v, [idx], ps[v])
            return carry

        lax.fori_loop(0, rpw, row, 0)
        pltpu.sync_copy(w_v, w_hbm.at[pl.ds(w * rpw * VP, rpw * VP)])

    return hop


def _make_count(B):
    rpw = B // NW
    nvec = TPAD // LANES

    @functools.partial(
        pl.kernel,
        mesh=_mesh(),
        out_type=jax.ShapeDtypeStruct((B * VP,), jnp.float32),
        scratch_types=[
            pltpu.VMEM((rpw * TPAD,), jnp.int32),
            pltpu.VMEM((rpw * VP,), jnp.float32),
        ],
        compiler_params=pltpu.CompilerParams(use_tc_tiling_on_sc=False, needs_layout_passes=False),
    )
    def count(q_hbm, c_hbm, q_v, c_v):
        w = _wid()
        pltpu.sync_copy(q_hbm.at[pl.ds(w * rpw * TPAD, rpw * TPAD)], q_v)
        ones = jnp.full((LANES,), 1.0, jnp.float32)

        def row(r, carry):
            for c in range(VP // LANES):
                c_v[pl.ds(r * VP + c * LANES, LANES)] = jnp.zeros((LANES,), jnp.float32)
            for v in range(nvec):
                idx = q_v[pl.ds(r * TPAD + v * LANES, LANES)]
                plsc.addupdate_scatter(c_v, [idx], ones)
            return carry

        lax.fori_loop(0, rpw, row, 0)
        pltpu.sync_copy(c_v, c_hbm.at[pl.ds(w * rpw * VP, rpw * VP)])

    return count


def _stage_body(w_ref, u_ref, a_ref, s_ref, unew_ref):
    a = a_ref[...]
    u = u_ref[...] + jax.lax.dot(
        w_ref[...], a, precision=_HIGH, preferred_element_type=jnp.float32)
    unew_ref[...] = u
    s_ref[...] = jax.lax.dot_general(
        u, a, (((1,), (1,)), ((), ())), precision=_HIGH,
        preferred_element_type=jnp.float32)


def _final_body(v_real, w_ref, u_ref, a_ref, out_ref):
    a = a_ref[...]
    u = u_ref[...] + jax.lax.dot(
        w_ref[...], a, precision=_HIGH, preferred_element_type=jnp.float32)
    logits = jax.lax.dot_general(
        u, a, (((1,), (1,)), ((), ())), precision=_HIGH,
        preferred_element_type=jnp.float32)
    col = lax.broadcasted_iota(jnp.int32, logits.shape, 1)
    logits = jnp.where(col < v_real, logits, -jnp.inf)
    mx = jnp.max(logits, axis=1, keepdims=True)
    e = jnp.exp(logits - mx)
    p = e / jnp.sum(e, axis=1, keepdims=True)
    out_ref[...] = lax.slice(p, (0, 0), (p.shape[0], v_real))


def kernel(x, query, A0, A1, A2, A3):
    B, story, sent = x.shape
    _, qlen = query.shape
    V, E = A0.shape
    rpw = B // NW

    roff = (jnp.arange(B, dtype=jnp.int32) % rpw) * VP
    xT = jnp.transpose(x, (0, 2, 1))
    xT = jnp.pad(xT, ((0, 0), (0, 0), (0, SPAD - story)))
    xoff = (xT + roff[:, None, None]).reshape(-1)
    qoff = (jnp.pad(query, ((0, 0), (0, TPAD - qlen))) + roff[:, None]).reshape(-1)
    Ap = [jnp.pad(A, ((0, VP - V), (0, 0))) for A in (A0, A1, A2, A3)]

    hop = _make_hop(B, story, sent)
    count = _make_count(B)
    stage = pl.pallas_call(
        _stage_body,
        out_shape=[
            jax.ShapeDtypeStruct((B, VP), jnp.float32),
            jax.ShapeDtypeStruct((B, E), jnp.float32),
        ],
    )
    final = pl.pallas_call(
        functools.partial(_final_body, V),
        out_shape=jax.ShapeDtypeStruct((B, V), jnp.float32),
    )

    Cq = count(qoff).reshape(B, VP)
    u = jnp.zeros((B, E), jnp.float32)
    W = Cq
    for k in range(3):
        S, u = stage(W, u, Ap[k])
        W = hop(S.reshape(-1), xoff).reshape(B, VP)
    return final(W, u, Ap[3])


# speedup vs baseline: 1.3196x; 1.1400x over previous
import functools

import jax
import jax.numpy as jnp
from jax import lax
from jax.experimental import pallas as pl
from jax.experimental.pallas import tpu as pltpu
from jax.experimental.pallas import tpu_sc as plsc

NC = 2
NS = 16
NW = NC * NS
LANES = 16

VP = 1024
SPAD = 64
TPAD = 32

_HIGH = jax.lax.Precision.HIGHEST


def _mesh():
    return plsc.VectorSubcoreMesh(
        core_axis_name="c", subcore_axis_name="s", num_cores=NC, num_subcores=NS)


def _wid():
    return lax.axis_index("s") * NC + lax.axis_index("c")


def _make_hop(B, story, T):
    rpw = B // NW
    nvec = SPAD // LANES
    n_real_last = story - (nvec - 1) * LANES
    xw = T * SPAD

    @functools.partial(
        pl.kernel,
        mesh=_mesh(),
        out_type=jax.ShapeDtypeStruct((B * VP,), jnp.float32),
        scratch_types=[
            pltpu.VMEM((rpw * VP,), jnp.float32),
            pltpu.VMEM((rpw * xw,), jnp.int32),
            pltpu.VMEM((rpw * VP,), jnp.float32),
            pltpu.SemaphoreType.DMA,
            pltpu.SemaphoreType.DMA,
        ],
        compiler_params=pltpu.CompilerParams(use_tc_tiling_on_sc=False, needs_layout_passes=False),
    )
    def hop(s_hbm, x_hbm, w_hbm, s_v, x_v, w_v, sem_s, sem_x):
        w = _wid()
        cp_s = pltpu.async_copy(s_hbm.at[pl.ds(w * rpw * VP, rpw * VP)], s_v, sem_s)
        cp_x = pltpu.async_copy(x_hbm.at[pl.ds(w * rpw * xw, rpw * xw)], x_v, sem_x)
        cp_s.wait()
        cp_x.wait()

        lane = lax.iota(jnp.int32, LANES)
        neg_inf = jnp.full((LANES,), -jnp.inf, jnp.float32)

        def row(r, carry):
            wbase = r * VP
            xbase = r * xw
            for c in range(VP // LANES):
                w_v[pl.ds(wbase + c * LANES, LANES)] = jnp.zeros((LANES,), jnp.float32)
            accs = [jnp.zeros((LANES,), jnp.float32) for _ in range(nvec)]
            for t in range(T):
                for v in range(nvec):
                    idx = x_v[pl.ds(xbase + t * SPAD + v * LANES, LANES)]
                    accs[v] = accs[v] + plsc.load_gather(s_v, [idx])
            accs[-1] = jnp.where(lane < n_real_last, accs[-1], neg_inf)
            m = accs[0]
            for v in range(1, nvec):
                m = jnp.maximum(m, accs[v])
            mx = jnp.max(m)
            es = [jnp.exp(a - jnp.full((LANES,), mx, jnp.float32)) for a in accs]
            tot = es[0]
            for v in range(1, nvec):
                tot = tot + es[v]
            totv = jnp.full((LANES,), jnp.sum(tot), jnp.float32)
            ps = [e / totv for e in es]
            pairs = [(t, v) for t in range(T) for v in range(nvec)]
            G = 8
            for g in range(0, len(pairs), G):
                grp = pairs[g:g + G]
                idxs = [
                    x_v[pl.ds(xbase + t * SPAD + v * LANES, LANES)]
                    for (t, v) in grp
                ]
                for (t, v), idx in zip(grp, idxs):
                    plsc.addupdate_scatter(w_v, [idx], ps[v])
            return carry

        lax.fori_loop(0, rpw, row, 0)
        pltpu.sync_copy(w_v, w_hbm.at[pl.ds(w * rpw * VP, rpw * VP)])

    return hop


def _make_count(B):
    rpw = B // NW
    nvec = TPAD // LANES

    @functools.partial(
        pl.kernel,
        mesh=_mesh(),
        out_type=jax.ShapeDtypeStruct((B * VP,), jnp.float32),
        scratch_types=[
            pltpu.VMEM((rpw * TPAD,), jnp.int32),
            pltpu.VMEM((rpw * VP,), jnp.float32),
        ],
        compiler_params=pltpu.CompilerParams(use_tc_tiling_on_sc=False, needs_layout_passes=False),
    )
    def count(q_hbm, c_hbm, q_v, c_v):
        w = _wid()
        pltpu.sync_copy(q_hbm.at[pl.ds(w * rpw * TPAD, rpw * TPAD)], q_v)
        ones = jnp.full((LANES,), 1.0, jnp.float32)

        def row(r, carry):
            for c in range(VP // LANES):
                c_v[pl.ds(r * VP + c * LANES, LANES)] = jnp.zeros((LANES,), jnp.float32)
            for v in range(nvec):
                idx = q_v[pl.ds(r * TPAD + v * LANES, LANES)]
                plsc.addupdate_scatter(c_v, [idx], ones)
            return carry

        lax.fori_loop(0, rpw, row, 0)
        pltpu.sync_copy(c_v, c_hbm.at[pl.ds(w * rpw * VP, rpw * VP)])

    return count


def _stage_body(w_ref, u_ref, a_ref, s_ref, unew_ref):
    a = a_ref[...]
    u = u_ref[...] + jax.lax.dot(
        w_ref[...], a, precision=_HIGH, preferred_element_type=jnp.float32)
    unew_ref[...] = u
    s_ref[...] = jax.lax.dot_general(
        u, a, (((1,), (1,)), ((), ())), precision=_HIGH,
        preferred_element_type=jnp.float32)


def _final_body(v_real, w_ref, u_ref, a_ref, out_ref):
    a = a_ref[...]
    u = u_ref[...] + jax.lax.dot(
        w_ref[...], a, precision=_HIGH, preferred_element_type=jnp.float32)
    logits = jax.lax.dot_general(
        u, a, (((1,), (1,)), ((), ())), precision=_HIGH,
        preferred_element_type=jnp.float32)
    col = lax.broadcasted_iota(jnp.int32, logits.shape, 1)
    logits = jnp.where(col < v_real, logits, -jnp.inf)
    mx = jnp.max(logits, axis=1, keepdims=True)
    e = jnp.exp(logits - mx)
    p = e / jnp.sum(e, axis=1, keepdims=True)
    out_ref[...] = lax.slice(p, (0, 0), (p.shape[0], v_real))


def kernel(x, query, A0, A1, A2, A3):
    B, story, sent = x.shape
    _, qlen = query.shape
    V, E = A0.shape
    rpw = B // NW

    roff = (jnp.arange(B, dtype=jnp.int32) % rpw) * VP
    xT = jnp.transpose(x, (0, 2, 1))
    xT = jnp.pad(xT, ((0, 0), (0, 0), (0, SPAD - story)))
    xoff = (xT + roff[:, None, None]).reshape(-1)
    qoff = (jnp.pad(query, ((0, 0), (0, TPAD - qlen))) + roff[:, None]).reshape(-1)
    Ap = [jnp.pad(A, ((0, VP - V), (0, 0))) for A in (A0, A1, A2, A3)]

    hop = _make_hop(B, story, sent)
    count = _make_count(B)
    stage = pl.pallas_call(
        _stage_body,
        out_shape=[
            jax.ShapeDtypeStruct((B, VP), jnp.float32),
            jax.ShapeDtypeStruct((B, E), jnp.float32),
        ],
    )
    final = pl.pallas_call(
        functools.partial(_final_body, V),
        out_shape=jax.ShapeDtypeStruct((B, V), jnp.float32),
    )

    Cq = count(qoff).reshape(B, VP)
    u = jnp.zeros((B, E), jnp.float32)
    W = Cq
    for k in range(3):
        S, u = stage(W, u, Ap[k])
        W = hop(S.reshape(-1), xoff).reshape(B, VP)
    return final(W, u, Ap[3])
